# Initial kernel scaffold; baseline (speedup 1.0000x reference)
import jax
import jax.numpy as jnp
from jax.experimental import pallas as pl


def kernel(x, edge_index, batch, W0, b0, W1, b1, W2, b2, W3, b3, Wbn0, bbn0, Wg0, bg0, Wbn1, bbn1, Wg1, bg1, Wbn2, bbn2, Wg2, bg2, Wbn3, bbn3, Wg3, bg3, Wend, bend):
    raise NotImplementedError("write your pallas kernel here")


# speedup vs baseline: 2.4368x; 2.4368x over previous
import jax
import jax.numpy as jnp
from jax.experimental import pallas as pl

_C = 126
_FEAT = 40
_TOPK = 20
_OFFS = (0, 128, 384, 640)
_WIDTHS = (128, 192, 256, 320)
_TOTF = 960


def _conv_as_matmul(W, b):
    O, _, R, kw = W.shape
    Wd = 10 - kw + 1
    A = jnp.zeros((R, 10, O, Wd), jnp.float32)
    Wt = jnp.transpose(W[:, 0, :, :], (1, 2, 0))
    for w in range(Wd):
        A = A.at[:, w:w + kw, :, w].set(Wt)
    return A.reshape(R * 10, O * Wd), jnp.repeat(b, Wd)


def _topk_mask(raw):
    lane = jax.lax.broadcasted_iota(jnp.int32, raw.shape, 1)

    def body(_, carry):
        a, m = carry
        rmax = jnp.max(a, axis=1, keepdims=True)
        first = jnp.min(jnp.where(a == rmax, lane, _C + 1), axis=1,
                        keepdims=True)
        sel = lane == first
        return jnp.where(sel, -1e30, a), jnp.where(sel, 1.0, m)

    _, mask = jax.lax.fori_loop(0, _TOPK, body,
                                (raw, jnp.zeros_like(raw)))
    return mask


def _fused_kernel(x_ref, a_ref, bf_ref,
                  wbn0, bbn0, wg0, bg0, wbn1, bbn1, wg1, bg1,
                  wbn2, bbn2, wg2, bg2, wbn3, bbn3, wg3, bg3,
                  wend_ref, bend_ref, lo_ref, pr_ref):
    xg = x_ref[0]
    feat = jnp.maximum(
        jnp.dot(xg, a_ref[...], preferred_element_type=jnp.float32)
        + bf_ref[0], 0.0)
    eye = (jax.lax.broadcasted_iota(jnp.int32, (_C, _C), 0)
           == jax.lax.broadcasted_iota(jnp.int32, (_C, _C), 1))
    acc = [jnp.zeros((1, 1), jnp.float32) for _ in range(3)]
    branch_w = ((wbn0, bbn0, wg0, bg0), (wbn1, bbn1, wg1, bg1),
                (wbn2, bbn2, wg2, bg2), (wbn3, bbn3, wg3, bg3))
    for i in range(4):
        wbn, bbn, wg, bg = branch_w[i]
        f = feat[:, _OFFS[i]:_OFFS[i] + _WIDTHS[i]]
        xa = jnp.tanh(
            jnp.dot(f, wbn[...], preferred_element_type=jnp.float32)
            + bbn[0])
        h = jnp.dot(f, wg[...], preferred_element_type=jnp.float32)
        raw = jax.lax.dot_general(xa, xa, (((1,), (1,)), ((), ())),
                                  preferred_element_type=jnp.float32)
        mask = _topk_mask(raw)
        rmax = jnp.max(raw, axis=1, keepdims=True)
        e = jnp.exp(raw - rmax)
        soft = e / jnp.sum(e, axis=1, keepdims=True)
        a2 = jnp.where(eye, 1.0, soft * mask)
        d = jax.lax.rsqrt(jnp.maximum(jnp.sum(a2, axis=1, keepdims=True),
                                      1.0))
        out = jnp.maximum(
            d * jnp.dot(a2, h * d, preferred_element_type=jnp.float32)
            + bg[0], 0.0)
        for t in range(3):
            p = out * wend_ref[t, i * _C:(i + 1) * _C, :]
            acc[t] = acc[t] + jnp.sum(jnp.sum(p, axis=1, keepdims=True),
                                      axis=0, keepdims=True)
    logits = jnp.concatenate(acc, axis=1) + bend_ref[...]
    lo_ref[0] = logits
    m = jnp.max(logits, axis=1, keepdims=True)
    ee = jnp.exp(logits - m)
    pr_ref[0] = ee / jnp.sum(ee, axis=1, keepdims=True)


def kernel(x, edge_index, batch, W0, b0, W1, b1, W2, b2, W3, b3,
           Wbn0, bbn0, Wg0, bg0, Wbn1, bbn1, Wg1, bg1,
           Wbn2, bbn2, Wg2, bg2, Wbn3, bbn3, Wg3, bg3,
           Wend, bend):
    n = x.shape[0] // _C
    xg = x.reshape(n, _C, _FEAT)
    A0, bf0 = _conv_as_matmul(W0, b0)
    A1, bf1 = _conv_as_matmul(W1, b1)
    A2, bf2 = _conv_as_matmul(W2, b2)
    A3, bf3 = _conv_as_matmul(W3, b3)
    zpadA = jnp.zeros((_FEAT, 64), jnp.float32)
    zpadb = jnp.zeros((64,), jnp.float32)
    A = jnp.concatenate([A0, A1, zpadA, A2, A3], axis=1)
    bf = jnp.concatenate([bf0, bf1, zpadb, bf2, bf3]).reshape(1, _TOTF)
    wend_r = Wend.reshape(4 * _C, 32, 3).transpose(2, 0, 1)

    def _full(shape):
        nd = len(shape)
        return pl.BlockSpec(shape, lambda g, _nd=nd: (0,) * _nd)

    weights = [A, bf,
               Wbn0, bbn0.reshape(1, 64), Wg0, bg0.reshape(1, 32),
               Wbn1, bbn1.reshape(1, 64), Wg1, bg1.reshape(1, 32),
               Wbn2, bbn2.reshape(1, 64), Wg2, bg2.reshape(1, 32),
               Wbn3, bbn3.reshape(1, 64), Wg3, bg3.reshape(1, 32),
               wend_r, bend.reshape(1, 3)]
    in_specs = [pl.BlockSpec((1, _C, _FEAT), lambda g: (g, 0, 0))]
    in_specs += [_full(w.shape) for w in weights]
    out_specs = [pl.BlockSpec((1, 1, 3), lambda g: (g, 0, 0))] * 2
    out_shape = [jax.ShapeDtypeStruct((n, 1, 3), jnp.float32)] * 2
    lo, pr = pl.pallas_call(
        _fused_kernel,
        grid=(n,),
        in_specs=in_specs,
        out_specs=out_specs,
        out_shape=out_shape,
    )(xg, *weights)
    return lo.reshape(n, 3), pr.reshape(n, 3)


# speedup vs baseline: 12.3433x; 5.0653x over previous
import jax
import jax.numpy as jnp
from jax.experimental import pallas as pl

_C = 126
_FEAT = 40
_TOPK = 20
_OFFS = (0, 128, 384, 640)
_WIDTHS = (128, 192, 256, 320)
_TOTF = 960
_SENT = -1e30


def _conv_as_matmul(W, b):
    O, _, R, kw = W.shape
    Wd = 10 - kw + 1
    A = jnp.zeros((R, 10, O, Wd), jnp.float32)
    Wt = jnp.transpose(W[:, 0, :, :], (1, 2, 0))
    for w in range(Wd):
        A = A.at[:, w:w + kw, :, w].set(Wt)
    return A.reshape(R * 10, O * Wd), jnp.repeat(b, Wd)


def _extract_pair(rawA, rawB):
    riota = jax.lax.broadcasted_iota(jnp.int32, (_C, _C), 0)

    def one(a):
        cmax = jnp.max(a, axis=0, keepdims=True)
        first = jnp.min(jnp.where(a == cmax, riota, _C + 1), axis=0,
                        keepdims=True)
        return jnp.where(riota == first, _SENT, a)

    def body(_, ab):
        return one(ab[0]), one(ab[1])

    a, b = jax.lax.fori_loop(0, _TOPK, body, (rawA, rawB))
    return a <= _SENT * 0.5, b <= _SENT * 0.5


def _fused_kernel(x_ref, a_ref, bf_ref,
                  wbn0, bbn0, wg0, bg0, wbn1, bbn1, wg1, bg1,
                  wbn2, bbn2, wg2, bg2, wbn3, bbn3, wg3, bg3,
                  wend_ref, bend_ref, lo_ref, pr_ref):
    xg = x_ref[0]
    feat = jnp.maximum(
        jnp.dot(xg, a_ref[...], preferred_element_type=jnp.float32)
        + bf_ref[0], 0.0)
    eye = (jax.lax.broadcasted_iota(jnp.int32, (_C, _C), 0)
           == jax.lax.broadcasted_iota(jnp.int32, (_C, _C), 1))
    eyef = jnp.where(eye, 1.0, 0.0)
    branch_w = ((wbn0, bbn0, wg0, bg0), (wbn1, bbn1, wg1, bg1),
                (wbn2, bbn2, wg2, bg2), (wbn3, bbn3, wg3, bg3))
    raws, hs = [], []
    for i in range(4):
        wbn, bbn, wg, bg = branch_w[i]
        f = feat[:, _OFFS[i]:_OFFS[i] + _WIDTHS[i]]
        xa = jnp.tanh(
            jnp.dot(f, wbn[...], preferred_element_type=jnp.float32)
            + bbn[0])
        hs.append(jnp.dot(f, wg[...], preferred_element_type=jnp.float32))
        raws.append(jax.lax.dot_general(
            xa, xa, (((1,), (1,)), ((), ())),
            preferred_element_type=jnp.float32))
    m0, m1 = _extract_pair(raws[0], raws[1])
    m2, m3 = _extract_pair(raws[2], raws[3])
    masks = (m0, m1, m2, m3)
    acc = [jnp.zeros((1, 32), jnp.float32) for _ in range(3)]
    for i in range(4):
        raw, h, mb = raws[i], hs[i], masks[i]
        bg = branch_w[i][3]
        cmax = jnp.max(raw, axis=0, keepdims=True)
        e = jnp.exp(raw - cmax)
        st = e / jnp.sum(e, axis=0, keepdims=True)
        P = jnp.where(eye, 1.0, jnp.where(mb, st, 0.0))
        deg = jnp.sum(P, axis=0, keepdims=True)
        d = jax.lax.rsqrt(jnp.maximum(deg, 1.0))
        dh = jnp.dot(eyef * d, h, preferred_element_type=jnp.float32)
        y = jax.lax.dot_general(P * d, dh, (((0,), (0,)), ((), ())),
                                preferred_element_type=jnp.float32)
        out = jnp.maximum(y + bg[0], 0.0)
        for t in range(3):
            p = out * wend_ref[t, i * _C:(i + 1) * _C, :]
            acc[t] = acc[t] + jnp.sum(p, axis=0, keepdims=True)
    lg = [jnp.sum(acc[t], axis=1, keepdims=True) for t in range(3)]
    logits = jnp.concatenate(lg, axis=1) + bend_ref[...]
    lo_ref[0] = logits
    m = jnp.max(logits, axis=1, keepdims=True)
    ee = jnp.exp(logits - m)
    pr_ref[0] = ee / jnp.sum(ee, axis=1, keepdims=True)


def kernel(x, edge_index, batch, W0, b0, W1, b1, W2, b2, W3, b3,
           Wbn0, bbn0, Wg0, bg0, Wbn1, bbn1, Wg1, bg1,
           Wbn2, bbn2, Wg2, bg2, Wbn3, bbn3, Wg3, bg3,
           Wend, bend):
    n = x.shape[0] // _C
    xg = x.reshape(n, _C, _FEAT)
    A0, bf0 = _conv_as_matmul(W0, b0)
    A1, bf1 = _conv_as_matmul(W1, b1)
    A2, bf2 = _conv_as_matmul(W2, b2)
    A3, bf3 = _conv_as_matmul(W3, b3)
    zpadA = jnp.zeros((_FEAT, 64), jnp.float32)
    zpadb = jnp.zeros((64,), jnp.float32)
    A = jnp.concatenate([A0, A1, zpadA, A2, A3], axis=1)
    bf = jnp.concatenate([bf0, bf1, zpadb, bf2, bf3]).reshape(1, _TOTF)
    wend_r = Wend.reshape(4 * _C, 32, 3).transpose(2, 0, 1)

    def _full(shape):
        nd = len(shape)
        return pl.BlockSpec(shape, lambda g, _nd=nd: (0,) * _nd)

    weights = [A, bf,
               Wbn0, bbn0.reshape(1, 64), Wg0, bg0.reshape(1, 32),
               Wbn1, bbn1.reshape(1, 64), Wg1, bg1.reshape(1, 32),
               Wbn2, bbn2.reshape(1, 64), Wg2, bg2.reshape(1, 32),
               Wbn3, bbn3.reshape(1, 64), Wg3, bg3.reshape(1, 32),
               wend_r, bend.reshape(1, 3)]
    in_specs = [pl.BlockSpec((1, _C, _FEAT), lambda g: (g, 0, 0))]
    in_specs += [_full(w.shape) for w in weights]
    out_specs = [pl.BlockSpec((1, 1, 3), lambda g: (g, 0, 0))] * 2
    out_shape = [jax.ShapeDtypeStruct((n, 1, 3), jnp.float32)] * 2
    lo, pr = pl.pallas_call(
        _fused_kernel,
        grid=(n,),
        in_specs=in_specs,
        out_specs=out_specs,
        out_shape=out_shape,
    )(xg, *weights)
    return lo.reshape(n, 3), pr.reshape(n, 3)


# speedup vs baseline: 20.4604x; 1.6576x over previous
import jax
import jax.numpy as jnp
from jax.experimental import pallas as pl

_C = 126
_FEAT = 40
_TOPK = 20
_OFFS = (0, 128, 384, 640)
_WIDTHS = (128, 192, 256, 320)
_TOTF = 960
_SENT = -1e30


def _conv_as_matmul(W, b):
    O, _, R, kw = W.shape
    Wd = 10 - kw + 1
    A = jnp.zeros((R, 10, O, Wd), jnp.float32)
    Wt = jnp.transpose(W[:, 0, :, :], (1, 2, 0))
    for w in range(Wd):
        A = A.at[:, w:w + kw, :, w].set(Wt)
    return A.reshape(R * 10, O * Wd), jnp.repeat(b, Wd)


_ISENT = -(2 ** 31)


def _pack_keys(raw):
    riota = jax.lax.broadcasted_iota(jnp.int32, (_C, _C), 0)
    bits = jax.lax.bitcast_convert_type(raw, jnp.int32)
    key0 = jnp.where(bits >= 0, bits, bits ^ jnp.int32(0x7FFFFFFF))
    return (key0 & jnp.int32(~0x7F)) | (jnp.int32(127) - riota)


def _extract_pair(kA, kB):
    for _ in range(_TOPK):
        mA = jnp.max(kA, axis=0, keepdims=True)
        mB = jnp.max(kB, axis=0, keepdims=True)
        kA = jnp.where(kA == mA, _ISENT, kA)
        kB = jnp.where(kB == mB, _ISENT, kB)
    return kA == _ISENT, kB == _ISENT


def _fused_kernel(x_ref, a_ref, bf_ref,
                  wbn0, bbn0, wg0, bg0, wbn1, bbn1, wg1, bg1,
                  wbn2, bbn2, wg2, bg2, wbn3, bbn3, wg3, bg3,
                  wend_ref, bend_ref, lo_ref, pr_ref):
    xg = x_ref[0]
    feat = jnp.maximum(
        jnp.dot(xg, a_ref[...], preferred_element_type=jnp.float32)
        + bf_ref[0], 0.0)
    eye = (jax.lax.broadcasted_iota(jnp.int32, (_C, _C), 0)
           == jax.lax.broadcasted_iota(jnp.int32, (_C, _C), 1))
    eyef = jnp.where(eye, 1.0, 0.0)
    branch_w = ((wbn0, bbn0, wg0, bg0), (wbn1, bbn1, wg1, bg1),
                (wbn2, bbn2, wg2, bg2), (wbn3, bbn3, wg3, bg3))
    raws, hs = [], []
    for i in range(4):
        wbn, bbn, wg, bg = branch_w[i]
        f = feat[:, _OFFS[i]:_OFFS[i] + _WIDTHS[i]]
        xa = jnp.tanh(
            jnp.dot(f, wbn[...], preferred_element_type=jnp.float32)
            + bbn[0])
        hs.append(jnp.dot(f, wg[...], preferred_element_type=jnp.float32))
        raws.append(jax.lax.dot_general(
            xa, xa, (((1,), (1,)), ((), ())),
            preferred_element_type=jnp.float32))
    m0, m1 = _extract_pair(_pack_keys(raws[0]), _pack_keys(raws[1]))
    m2, m3 = _extract_pair(_pack_keys(raws[2]), _pack_keys(raws[3]))
    masks = (m0, m1, m2, m3)
    acc = [jnp.zeros((1, 32), jnp.float32) for _ in range(3)]
    for i in range(4):
        raw, h, mb = raws[i], hs[i], masks[i]
        bg = branch_w[i][3]
        cmax = jnp.max(raw, axis=0, keepdims=True)
        e = jnp.exp(raw - cmax)
        st = e / jnp.sum(e, axis=0, keepdims=True)
        P = jnp.where(eye, 1.0, jnp.where(mb, st, 0.0))
        deg = jnp.sum(P, axis=0, keepdims=True)
        d = jax.lax.rsqrt(jnp.maximum(deg, 1.0))
        dh = jnp.dot(eyef * d, h, preferred_element_type=jnp.float32)
        y = jax.lax.dot_general(P * d, dh, (((0,), (0,)), ((), ())),
                                preferred_element_type=jnp.float32)
        out = jnp.maximum(y + bg[0], 0.0)
        for t in range(3):
            p = out * wend_ref[t, i * _C:(i + 1) * _C, :]
            acc[t] = acc[t] + jnp.sum(p, axis=0, keepdims=True)
    lg = [jnp.sum(acc[t], axis=1, keepdims=True) for t in range(3)]
    logits = jnp.concatenate(lg, axis=1) + bend_ref[...]
    lo_ref[0] = logits
    m = jnp.max(logits, axis=1, keepdims=True)
    ee = jnp.exp(logits - m)
    pr_ref[0] = ee / jnp.sum(ee, axis=1, keepdims=True)


def kernel(x, edge_index, batch, W0, b0, W1, b1, W2, b2, W3, b3,
           Wbn0, bbn0, Wg0, bg0, Wbn1, bbn1, Wg1, bg1,
           Wbn2, bbn2, Wg2, bg2, Wbn3, bbn3, Wg3, bg3,
           Wend, bend):
    n = x.shape[0] // _C
    xg = x.reshape(n, _C, _FEAT)
    A0, bf0 = _conv_as_matmul(W0, b0)
    A1, bf1 = _conv_as_matmul(W1, b1)
    A2, bf2 = _conv_as_matmul(W2, b2)
    A3, bf3 = _conv_as_matmul(W3, b3)
    zpadA = jnp.zeros((_FEAT, 64), jnp.float32)
    zpadb = jnp.zeros((64,), jnp.float32)
    A = jnp.concatenate([A0, A1, zpadA, A2, A3], axis=1)
    bf = jnp.concatenate([bf0, bf1, zpadb, bf2, bf3]).reshape(1, _TOTF)
    wend_r = Wend.reshape(4 * _C, 32, 3).transpose(2, 0, 1)

    def _full(shape):
        nd = len(shape)
        return pl.BlockSpec(shape, lambda g, _nd=nd: (0,) * _nd)

    weights = [A, bf,
               Wbn0, bbn0.reshape(1, 64), Wg0, bg0.reshape(1, 32),
               Wbn1, bbn1.reshape(1, 64), Wg1, bg1.reshape(1, 32),
               Wbn2, bbn2.reshape(1, 64), Wg2, bg2.reshape(1, 32),
               Wbn3, bbn3.reshape(1, 64), Wg3, bg3.reshape(1, 32),
               wend_r, bend.reshape(1, 3)]
    in_specs = [pl.BlockSpec((1, _C, _FEAT), lambda g: (g, 0, 0))]
    in_specs += [_full(w.shape) for w in weights]
    out_specs = [pl.BlockSpec((1, 1, 3), lambda g: (g, 0, 0))] * 2
    out_shape = [jax.ShapeDtypeStruct((n, 1, 3), jnp.float32)] * 2
    lo, pr = pl.pallas_call(
        _fused_kernel,
        grid=(n,),
        in_specs=in_specs,
        out_specs=out_specs,
        out_shape=out_shape,
    )(xg, *weights)
    return lo.reshape(n, 3), pr.reshape(n, 3)


# speedup vs baseline: 22.6689x; 1.1079x over previous
import jax
import jax.numpy as jnp
from jax.experimental import pallas as pl

_C = 126
_FEAT = 40
_TOPK = 20
_G = 2
_OFFS = (0, 128, 384, 640)
_WIDTHS = (128, 192, 256, 320)
_TOTF = 960
_ISENT = -(2 ** 31)


def _conv_as_matmul(W, b):
    O, _, R, kw = W.shape
    Wd = 10 - kw + 1
    A = jnp.zeros((R, 10, O, Wd), jnp.float32)
    Wt = jnp.transpose(W[:, 0, :, :], (1, 2, 0))
    for w in range(Wd):
        A = A.at[:, w:w + kw, :, w].set(Wt)
    return A.reshape(R * 10, O * Wd), jnp.repeat(b, Wd)


def _pack_keys(raw):
    riota = jax.lax.broadcasted_iota(jnp.int32, (_C, _C), 0)
    bits = jax.lax.bitcast_convert_type(raw, jnp.int32)
    key0 = jnp.where(bits >= 0, bits, bits ^ jnp.int32(0x7FFFFFFF))
    return (key0 & jnp.int32(~0x7F)) | (jnp.int32(127) - riota)


def _extract(keys):
    keys = list(keys)
    for _ in range(_TOPK):
        for j in range(len(keys)):
            m = jnp.max(keys[j], axis=0, keepdims=True)
            keys[j] = jnp.where(keys[j] == m, _ISENT, keys[j])
    return [k == _ISENT for k in keys]


def _graph_body(xg, a_ref, bf_ref, branch_w, wend_ref, eye, eyef):
    feat = jnp.maximum(
        jnp.dot(xg, a_ref[...], preferred_element_type=jnp.float32)
        + bf_ref[0], 0.0)
    raws, hs = [], []
    for i in range(4):
        wbn, bbn, wg, bg = branch_w[i]
        f = feat[:, _OFFS[i]:_OFFS[i] + _WIDTHS[i]]
        xa = jnp.tanh(
            jnp.dot(f, wbn[...], preferred_element_type=jnp.float32)
            + bbn[0])
        hs.append(jnp.dot(f, wg[...], preferred_element_type=jnp.float32))
        raws.append(jax.lax.dot_general(
            xa, xa, (((1,), (1,)), ((), ())),
            preferred_element_type=jnp.float32))
    masks = _extract([_pack_keys(r) for r in raws])
    acc = [jnp.zeros((1, 32), jnp.float32) for _ in range(3)]
    for i in range(4):
        raw, h, mb = raws[i], hs[i], masks[i]
        bg = branch_w[i][3]
        cmax = jnp.max(raw, axis=0, keepdims=True)
        e = jnp.exp(raw - cmax)
        st = e / jnp.sum(e, axis=0, keepdims=True)
        P = jnp.where(eye, 1.0, jnp.where(mb, st, 0.0))
        deg = jnp.sum(P, axis=0, keepdims=True)
        d = jax.lax.rsqrt(jnp.maximum(deg, 1.0))
        dcol = jax.lax.dot_general(eyef, d, (((1,), (1,)), ((), ())),
                                   preferred_element_type=jnp.float32)
        y = jax.lax.dot_general(P * d, h * dcol, (((0,), (0,)), ((), ())),
                                preferred_element_type=jnp.float32)
        out = jnp.maximum(y + bg[0], 0.0)
        for t in range(3):
            p = out * wend_ref[t, i * _C:(i + 1) * _C, :]
            acc[t] = acc[t] + jnp.sum(p, axis=0, keepdims=True)
    lg = [jnp.sum(acc[t], axis=1, keepdims=True) for t in range(3)]
    return jnp.concatenate(lg, axis=1)


def _fused_kernel(x_ref, a_ref, bf_ref,
                  wbn0, bbn0, wg0, bg0, wbn1, bbn1, wg1, bg1,
                  wbn2, bbn2, wg2, bg2, wbn3, bbn3, wg3, bg3,
                  wend_ref, bend_ref, lo_ref, pr_ref):
    eye = (jax.lax.broadcasted_iota(jnp.int32, (_C, _C), 0)
           == jax.lax.broadcasted_iota(jnp.int32, (_C, _C), 1))
    eyef = jnp.where(eye, 1.0, 0.0)
    branch_w = ((wbn0, bbn0, wg0, bg0), (wbn1, bbn1, wg1, bg1),
                (wbn2, bbn2, wg2, bg2), (wbn3, bbn3, wg3, bg3))
    for gidx in range(_G):
        logits = _graph_body(x_ref[gidx], a_ref, bf_ref, branch_w,
                             wend_ref, eye, eyef) + bend_ref[...]
        lo_ref[gidx] = logits
        m = jnp.max(logits, axis=1, keepdims=True)
        ee = jnp.exp(logits - m)
        pr_ref[gidx] = ee / jnp.sum(ee, axis=1, keepdims=True)


def kernel(x, edge_index, batch, W0, b0, W1, b1, W2, b2, W3, b3,
           Wbn0, bbn0, Wg0, bg0, Wbn1, bbn1, Wg1, bg1,
           Wbn2, bbn2, Wg2, bg2, Wbn3, bbn3, Wg3, bg3,
           Wend, bend):
    n = x.shape[0] // _C
    xg = x.reshape(n, _C, _FEAT)
    A0, bf0 = _conv_as_matmul(W0, b0)
    A1, bf1 = _conv_as_matmul(W1, b1)
    A2, bf2 = _conv_as_matmul(W2, b2)
    A3, bf3 = _conv_as_matmul(W3, b3)
    zpadA = jnp.zeros((_FEAT, 64), jnp.float32)
    zpadb = jnp.zeros((64,), jnp.float32)
    A = jnp.concatenate([A0, A1, zpadA, A2, A3], axis=1)
    bf = jnp.concatenate([bf0, bf1, zpadb, bf2, bf3]).reshape(1, _TOTF)
    wend_r = Wend.reshape(4 * _C, 32, 3).transpose(2, 0, 1)

    def _full(shape):
        nd = len(shape)
        return pl.BlockSpec(shape, lambda g, _nd=nd: (0,) * _nd)

    weights = [A, bf,
               Wbn0, bbn0.reshape(1, 64), Wg0, bg0.reshape(1, 32),
               Wbn1, bbn1.reshape(1, 64), Wg1, bg1.reshape(1, 32),
               Wbn2, bbn2.reshape(1, 64), Wg2, bg2.reshape(1, 32),
               Wbn3, bbn3.reshape(1, 64), Wg3, bg3.reshape(1, 32),
               wend_r, bend.reshape(1, 3)]
    in_specs = [pl.BlockSpec((_G, _C, _FEAT), lambda g: (g, 0, 0))]
    in_specs += [_full(w.shape) for w in weights]
    out_specs = [pl.BlockSpec((_G, 1, 3), lambda g: (g, 0, 0))] * 2
    out_shape = [jax.ShapeDtypeStruct((n, 1, 3), jnp.float32)] * 2
    lo, pr = pl.pallas_call(
        _fused_kernel,
        grid=(n // _G,),
        in_specs=in_specs,
        out_specs=out_specs,
        out_shape=out_shape,
    )(xg, *weights)
    return lo.reshape(n, 3), pr.reshape(n, 3)


# speedup vs baseline: 23.0849x; 1.0184x over previous
import jax
import jax.numpy as jnp
from jax.experimental import pallas as pl

_C = 126
_FEAT = 40
_TOPK = 20
_G = 2
_OFFS = (0, 128, 384, 640)
_WIDTHS = (128, 192, 256, 320)
_TOTF = 960
_ISENT = -(2 ** 31)


def _conv_as_matmul(W, b):
    O, _, R, kw = W.shape
    Wd = 10 - kw + 1
    A = jnp.zeros((R, 10, O, Wd), jnp.float32)
    Wt = jnp.transpose(W[:, 0, :, :], (1, 2, 0))
    for w in range(Wd):
        A = A.at[:, w:w + kw, :, w].set(Wt)
    return A.reshape(R * 10, O * Wd), jnp.repeat(b, Wd)


def _pack_keys(raw):
    riota = jax.lax.broadcasted_iota(jnp.int32, (_C, _C), 0)
    bits = jax.lax.bitcast_convert_type(raw, jnp.int32)
    key0 = jnp.where(bits >= 0, bits, bits ^ jnp.int32(0x7FFFFFFF))
    key = (((key0 >> 8) << 7) | (jnp.int32(127) - riota)) \
        + jnp.int32(0x30000000)
    return jax.lax.bitcast_convert_type(key, jnp.float32)


def _extract(keys):
    keys = list(keys)
    for _ in range(_TOPK):
        for j in range(len(keys)):
            m = jnp.max(keys[j], axis=0, keepdims=True)
            keys[j] = jnp.where(keys[j] == m, -1.0, keys[j])
    return [k < 0.0 for k in keys]


def _graph_body(xg, a_ref, bf_ref, branch_w, wend_ref, eye, eyef):
    feat = jnp.maximum(
        jnp.dot(xg, a_ref[...], preferred_element_type=jnp.float32)
        + bf_ref[0], 0.0)
    raws, hs = [], []
    for i in range(4):
        wbn, bbn, wg, bg = branch_w[i]
        f = feat[:, _OFFS[i]:_OFFS[i] + _WIDTHS[i]]
        xa = jnp.tanh(
            jnp.dot(f, wbn[...], preferred_element_type=jnp.float32)
            + bbn[0])
        hs.append(jnp.dot(f, wg[...], preferred_element_type=jnp.float32))
        raws.append(jax.lax.dot_general(
            xa, xa, (((1,), (1,)), ((), ())),
            preferred_element_type=jnp.float32))
    masks = _extract([_pack_keys(r) for r in raws])
    acc = [jnp.zeros((1, 32), jnp.float32) for _ in range(3)]
    for i in range(4):
        raw, h, mb = raws[i], hs[i], masks[i]
        bg = branch_w[i][3]
        cmax = jnp.max(raw, axis=0, keepdims=True)
        e = jnp.exp(raw - cmax)
        st = e / jnp.sum(e, axis=0, keepdims=True)
        P = jnp.where(eye, 1.0, jnp.where(mb, st, 0.0))
        deg = jnp.sum(P, axis=0, keepdims=True)
        d = jax.lax.rsqrt(jnp.maximum(deg, 1.0))
        dcol = jax.lax.dot_general(eyef, d, (((1,), (1,)), ((), ())),
                                   preferred_element_type=jnp.float32)
        y = jax.lax.dot_general(P * d, h * dcol, (((0,), (0,)), ((), ())),
                                preferred_element_type=jnp.float32)
        out = jnp.maximum(y + bg[0], 0.0)
        for t in range(3):
            p = out * wend_ref[t, i * _C:(i + 1) * _C, :]
            acc[t] = acc[t] + jnp.sum(p, axis=0, keepdims=True)
    lg = [jnp.sum(acc[t], axis=1, keepdims=True) for t in range(3)]
    return jnp.concatenate(lg, axis=1)


def _fused_kernel(x_ref, a_ref, bf_ref,
                  wbn0, bbn0, wg0, bg0, wbn1, bbn1, wg1, bg1,
                  wbn2, bbn2, wg2, bg2, wbn3, bbn3, wg3, bg3,
                  wend_ref, bend_ref, lo_ref, pr_ref):
    eye = (jax.lax.broadcasted_iota(jnp.int32, (_C, _C), 0)
           == jax.lax.broadcasted_iota(jnp.int32, (_C, _C), 1))
    eyef = jnp.where(eye, 1.0, 0.0)
    branch_w = ((wbn0, bbn0, wg0, bg0), (wbn1, bbn1, wg1, bg1),
                (wbn2, bbn2, wg2, bg2), (wbn3, bbn3, wg3, bg3))
    for gidx in range(_G):
        logits = _graph_body(x_ref[gidx], a_ref, bf_ref, branch_w,
                             wend_ref, eye, eyef) + bend_ref[...]
        lo_ref[gidx] = logits
        m = jnp.max(logits, axis=1, keepdims=True)
        ee = jnp.exp(logits - m)
        pr_ref[gidx] = ee / jnp.sum(ee, axis=1, keepdims=True)


def kernel(x, edge_index, batch, W0, b0, W1, b1, W2, b2, W3, b3,
           Wbn0, bbn0, Wg0, bg0, Wbn1, bbn1, Wg1, bg1,
           Wbn2, bbn2, Wg2, bg2, Wbn3, bbn3, Wg3, bg3,
           Wend, bend):
    n = x.shape[0] // _C
    xg = x.reshape(n, _C, _FEAT)
    A0, bf0 = _conv_as_matmul(W0, b0)
    A1, bf1 = _conv_as_matmul(W1, b1)
    A2, bf2 = _conv_as_matmul(W2, b2)
    A3, bf3 = _conv_as_matmul(W3, b3)
    zpadA = jnp.zeros((_FEAT, 64), jnp.float32)
    zpadb = jnp.zeros((64,), jnp.float32)
    A = jnp.concatenate([A0, A1, zpadA, A2, A3], axis=1)
    bf = jnp.concatenate([bf0, bf1, zpadb, bf2, bf3]).reshape(1, _TOTF)
    wend_r = Wend.reshape(4 * _C, 32, 3).transpose(2, 0, 1)

    def _full(shape):
        nd = len(shape)
        return pl.BlockSpec(shape, lambda g, _nd=nd: (0,) * _nd)

    weights = [A, bf,
               Wbn0, bbn0.reshape(1, 64), Wg0, bg0.reshape(1, 32),
               Wbn1, bbn1.reshape(1, 64), Wg1, bg1.reshape(1, 32),
               Wbn2, bbn2.reshape(1, 64), Wg2, bg2.reshape(1, 32),
               Wbn3, bbn3.reshape(1, 64), Wg3, bg3.reshape(1, 32),
               wend_r, bend.reshape(1, 3)]
    in_specs = [pl.BlockSpec((_G, _C, _FEAT), lambda g: (g, 0, 0))]
    in_specs += [_full(w.shape) for w in weights]
    out_specs = [pl.BlockSpec((_G, 1, 3), lambda g: (g, 0, 0))] * 2
    out_shape = [jax.ShapeDtypeStruct((n, 1, 3), jnp.float32)] * 2
    lo, pr = pl.pallas_call(
        _fused_kernel,
        grid=(n // _G,),
        in_specs=in_specs,
        out_specs=out_specs,
        out_shape=out_shape,
    )(xg, *weights)
    return lo.reshape(n, 3), pr.reshape(n, 3)


# speedup vs baseline: 24.6714x; 1.0687x over previous
import jax
import jax.numpy as jnp
from jax.experimental import pallas as pl

_C = 126
_FEAT = 40
_TOPK = 20
_G = 4
_OFFS = (0, 128, 384, 640)
_WIDTHS = (128, 192, 256, 320)
_TOTF = 960
_ISENT = -(2 ** 31)


def _conv_as_matmul(W, b):
    O, _, R, kw = W.shape
    Wd = 10 - kw + 1
    A = jnp.zeros((R, 10, O, Wd), jnp.float32)
    Wt = jnp.transpose(W[:, 0, :, :], (1, 2, 0))
    for w in range(Wd):
        A = A.at[:, w:w + kw, :, w].set(Wt)
    return A.reshape(R * 10, O * Wd), jnp.repeat(b, Wd)


def _pack_keys(raw):
    riota = jax.lax.broadcasted_iota(jnp.int32, (_C, _C), 0)
    bits = jax.lax.bitcast_convert_type(raw, jnp.int32)
    key0 = jnp.where(bits >= 0, bits, bits ^ jnp.int32(0x7FFFFFFF))
    key = (((key0 >> 8) << 7) | (jnp.int32(127) - riota)) \
        + jnp.int32(0x30000000)
    return jax.lax.bitcast_convert_type(key, jnp.float32)


def _extract(keys):
    keys = list(keys)
    for _ in range(_TOPK):
        for j in range(len(keys)):
            m = jnp.max(keys[j], axis=0, keepdims=True)
            keys[j] = jnp.where(keys[j] == m, -1.0, keys[j])
    return [k < 0.0 for k in keys]


def _graph_body(xg, a_ref, bf_ref, branch_w, wend_ref, eye, eyef):
    feat = jnp.maximum(
        jnp.dot(xg, a_ref[...], preferred_element_type=jnp.float32)
        + bf_ref[0], 0.0)
    raws, hs = [], []
    for i in range(4):
        wbn, bbn, wg, bg = branch_w[i]
        f = feat[:, _OFFS[i]:_OFFS[i] + _WIDTHS[i]]
        xa = jnp.tanh(
            jnp.dot(f, wbn[...], preferred_element_type=jnp.float32)
            + bbn[0])
        hs.append(jnp.dot(f, wg[...], preferred_element_type=jnp.float32))
        raws.append(jax.lax.dot_general(
            xa, xa, (((1,), (1,)), ((), ())),
            preferred_element_type=jnp.float32))
    masks = _extract([_pack_keys(r) for r in raws])
    acc = [jnp.zeros((1, 32), jnp.float32) for _ in range(3)]
    for i in range(4):
        raw, h, mb = raws[i], hs[i], masks[i]
        bg = branch_w[i][3]
        cmax = jnp.max(raw, axis=0, keepdims=True)
        e = jnp.exp(raw - cmax)
        st = e / jnp.sum(e, axis=0, keepdims=True)
        P = jnp.where(eye, 1.0, jnp.where(mb, st, 0.0))
        deg = jnp.sum(P, axis=0, keepdims=True)
        d = jax.lax.rsqrt(jnp.maximum(deg, 1.0))
        dcol = jax.lax.dot_general(eyef, d, (((1,), (1,)), ((), ())),
                                   preferred_element_type=jnp.float32)
        y = jax.lax.dot_general(P * d, h * dcol, (((0,), (0,)), ((), ())),
                                preferred_element_type=jnp.float32)
        out = jnp.maximum(y + bg[0], 0.0)
        for t in range(3):
            p = out * wend_ref[t, i * _C:(i + 1) * _C, :]
            acc[t] = acc[t] + jnp.sum(p, axis=0, keepdims=True)
    lg = [jnp.sum(acc[t], axis=1, keepdims=True) for t in range(3)]
    return jnp.concatenate(lg, axis=1)


def _fused_kernel(x_ref, a_ref, bf_ref,
                  wbn0, bbn0, wg0, bg0, wbn1, bbn1, wg1, bg1,
                  wbn2, bbn2, wg2, bg2, wbn3, bbn3, wg3, bg3,
                  wend_ref, bend_ref, lo_ref, pr_ref):
    eye = (jax.lax.broadcasted_iota(jnp.int32, (_C, _C), 0)
           == jax.lax.broadcasted_iota(jnp.int32, (_C, _C), 1))
    eyef = jnp.where(eye, 1.0, 0.0)
    branch_w = ((wbn0, bbn0, wg0, bg0), (wbn1, bbn1, wg1, bg1),
                (wbn2, bbn2, wg2, bg2), (wbn3, bbn3, wg3, bg3))
    for gidx in range(_G):
        logits = _graph_body(x_ref[gidx], a_ref, bf_ref, branch_w,
                             wend_ref, eye, eyef) + bend_ref[...]
        lo_ref[gidx] = logits
        m = jnp.max(logits, axis=1, keepdims=True)
        ee = jnp.exp(logits - m)
        pr_ref[gidx] = ee / jnp.sum(ee, axis=1, keepdims=True)


def kernel(x, edge_index, batch, W0, b0, W1, b1, W2, b2, W3, b3,
           Wbn0, bbn0, Wg0, bg0, Wbn1, bbn1, Wg1, bg1,
           Wbn2, bbn2, Wg2, bg2, Wbn3, bbn3, Wg3, bg3,
           Wend, bend):
    n = x.shape[0] // _C
    xg = x.reshape(n, _C, _FEAT)
    A0, bf0 = _conv_as_matmul(W0, b0)
    A1, bf1 = _conv_as_matmul(W1, b1)
    A2, bf2 = _conv_as_matmul(W2, b2)
    A3, bf3 = _conv_as_matmul(W3, b3)
    zpadA = jnp.zeros((_FEAT, 64), jnp.float32)
    zpadb = jnp.zeros((64,), jnp.float32)
    A = jnp.concatenate([A0, A1, zpadA, A2, A3], axis=1)
    bf = jnp.concatenate([bf0, bf1, zpadb, bf2, bf3]).reshape(1, _TOTF)
    wend_r = Wend.reshape(4 * _C, 32, 3).transpose(2, 0, 1)

    def _full(shape):
        nd = len(shape)
        return pl.BlockSpec(shape, lambda g, _nd=nd: (0,) * _nd)

    weights = [A, bf,
               Wbn0, bbn0.reshape(1, 64), Wg0, bg0.reshape(1, 32),
               Wbn1, bbn1.reshape(1, 64), Wg1, bg1.reshape(1, 32),
               Wbn2, bbn2.reshape(1, 64), Wg2, bg2.reshape(1, 32),
               Wbn3, bbn3.reshape(1, 64), Wg3, bg3.reshape(1, 32),
               wend_r, bend.reshape(1, 3)]
    in_specs = [pl.BlockSpec((_G, _C, _FEAT), lambda g: (g, 0, 0))]
    in_specs += [_full(w.shape) for w in weights]
    out_specs = [pl.BlockSpec((_G, 1, 3), lambda g: (g, 0, 0))] * 2
    out_shape = [jax.ShapeDtypeStruct((n, 1, 3), jnp.float32)] * 2
    lo, pr = pl.pallas_call(
        _fused_kernel,
        grid=(n // _G,),
        in_specs=in_specs,
        out_specs=out_specs,
        out_shape=out_shape,
    )(xg, *weights)
    return lo.reshape(n, 3), pr.reshape(n, 3)


# speedup vs baseline: 37.1868x; 1.5073x over previous
import jax
import jax.numpy as jnp
from jax.experimental import pallas as pl

_C = 126
_FEAT = 40
_TOPK = 20
_G = 4
_OFFS = (0, 128, 384, 640)
_WIDTHS = (128, 192, 256, 320)
_TOTF = 960
_ISENT = -(2 ** 31)


def _conv_as_matmul(W, b):
    O, _, R, kw = W.shape
    Wd = 10 - kw + 1
    A = jnp.zeros((R, 10, O, Wd), jnp.float32)
    Wt = jnp.transpose(W[:, 0, :, :], (1, 2, 0))
    for w in range(Wd):
        A = A.at[:, w:w + kw, :, w].set(Wt)
    return A.reshape(R * 10, O * Wd), jnp.repeat(b, Wd)


def _pack_keys(raw):
    riota = jax.lax.broadcasted_iota(jnp.int32, (_C, _C), 0)
    bits = jax.lax.bitcast_convert_type(raw, jnp.int32)
    key0 = jnp.where(bits >= 0, bits, bits ^ jnp.int32(0x7FFFFFFF))
    key = (((key0 >> 8) << 7) | (jnp.int32(127) - riota)) \
        + jnp.int32(0x30000000)
    return jax.lax.bitcast_convert_type(key, jnp.float32)


def _batcher16():
    pairs = []
    p = 1
    while p < 16:
        k = p
        while k >= 1:
            for j in range(k % p, 16 - k, 2 * k):
                for i in range(0, min(k, 16 - j - k)):
                    if (i + j) // (p * 2) == (i + j + k) // (p * 2):
                        pairs.append((i + j, i + j + k))
            k //= 2
        p *= 2
    return pairs


_NET16 = _batcher16()


def _topk_threshold(keyf):
    kp = jax.lax.pad(keyf, jnp.float32(-1.0), [(0, 2, 0), (0, 0, 0)])
    s = [kp[8 * i:8 * (i + 1), :] for i in range(16)]
    for (i, j) in _NET16:
        hi = jnp.maximum(s[i], s[j])
        s[j] = jnp.minimum(s[i], s[j])
        s[i] = hi
    for t in range(_TOPK - 1):
        m = jnp.max(s[0], axis=0, keepdims=True)
        sel = s[0] == m
        depth = _TOPK - t - 1
        for i in range(min(15, depth)):
            s[i] = jnp.where(sel, s[i + 1], s[i])
        if depth > 15:
            s[15] = jnp.where(sel, -1.0, s[15])
    return jnp.max(s[0], axis=0, keepdims=True)


def _extract(keys):
    return [k >= _topk_threshold(k) for k in keys]


def _graph_body(xg, a_ref, bf_ref, branch_w, wend_ref, eye, eyef):
    feat = jnp.maximum(
        jnp.dot(xg, a_ref[...], preferred_element_type=jnp.float32)
        + bf_ref[0], 0.0)
    raws, hs = [], []
    for i in range(4):
        wbn, bbn, wg, bg = branch_w[i]
        f = feat[:, _OFFS[i]:_OFFS[i] + _WIDTHS[i]]
        xa = jnp.tanh(
            jnp.dot(f, wbn[...], preferred_element_type=jnp.float32)
            + bbn[0])
        hs.append(jnp.dot(f, wg[...], preferred_element_type=jnp.float32))
        raws.append(jax.lax.dot_general(
            xa, xa, (((1,), (1,)), ((), ())),
            preferred_element_type=jnp.float32))
    masks = _extract([_pack_keys(r) for r in raws])
    acc = [jnp.zeros((1, 32), jnp.float32) for _ in range(3)]
    for i in range(4):
        raw, h, mb = raws[i], hs[i], masks[i]
        bg = branch_w[i][3]
        cmax = jnp.max(raw, axis=0, keepdims=True)
        e = jnp.exp(raw - cmax)
        st = e / jnp.sum(e, axis=0, keepdims=True)
        P = jnp.where(eye, 1.0, jnp.where(mb, st, 0.0))
        deg = jnp.sum(P, axis=0, keepdims=True)
        d = jax.lax.rsqrt(jnp.maximum(deg, 1.0))
        dcol = jax.lax.dot_general(eyef, d, (((1,), (1,)), ((), ())),
                                   preferred_element_type=jnp.float32)
        y = jax.lax.dot_general(P * d, h * dcol, (((0,), (0,)), ((), ())),
                                preferred_element_type=jnp.float32)
        out = jnp.maximum(y + bg[0], 0.0)
        for t in range(3):
            p = out * wend_ref[t, i * _C:(i + 1) * _C, :]
            acc[t] = acc[t] + jnp.sum(p, axis=0, keepdims=True)
    lg = [jnp.sum(acc[t], axis=1, keepdims=True) for t in range(3)]
    return jnp.concatenate(lg, axis=1)


def _fused_kernel(x_ref, a_ref, bf_ref,
                  wbn0, bbn0, wg0, bg0, wbn1, bbn1, wg1, bg1,
                  wbn2, bbn2, wg2, bg2, wbn3, bbn3, wg3, bg3,
                  wend_ref, bend_ref, lo_ref, pr_ref):
    eye = (jax.lax.broadcasted_iota(jnp.int32, (_C, _C), 0)
           == jax.lax.broadcasted_iota(jnp.int32, (_C, _C), 1))
    eyef = jnp.where(eye, 1.0, 0.0)
    branch_w = ((wbn0, bbn0, wg0, bg0), (wbn1, bbn1, wg1, bg1),
                (wbn2, bbn2, wg2, bg2), (wbn3, bbn3, wg3, bg3))
    for gidx in range(_G):
        logits = _graph_body(x_ref[gidx], a_ref, bf_ref, branch_w,
                             wend_ref, eye, eyef) + bend_ref[...]
        lo_ref[gidx] = logits
        m = jnp.max(logits, axis=1, keepdims=True)
        ee = jnp.exp(logits - m)
        pr_ref[gidx] = ee / jnp.sum(ee, axis=1, keepdims=True)


def kernel(x, edge_index, batch, W0, b0, W1, b1, W2, b2, W3, b3,
           Wbn0, bbn0, Wg0, bg0, Wbn1, bbn1, Wg1, bg1,
           Wbn2, bbn2, Wg2, bg2, Wbn3, bbn3, Wg3, bg3,
           Wend, bend):
    n = x.shape[0] // _C
    xg = x.reshape(n, _C, _FEAT)
    A0, bf0 = _conv_as_matmul(W0, b0)
    A1, bf1 = _conv_as_matmul(W1, b1)
    A2, bf2 = _conv_as_matmul(W2, b2)
    A3, bf3 = _conv_as_matmul(W3, b3)
    zpadA = jnp.zeros((_FEAT, 64), jnp.float32)
    zpadb = jnp.zeros((64,), jnp.float32)
    A = jnp.concatenate([A0, A1, zpadA, A2, A3], axis=1)
    bf = jnp.concatenate([bf0, bf1, zpadb, bf2, bf3]).reshape(1, _TOTF)
    wend_r = Wend.reshape(4 * _C, 32, 3).transpose(2, 0, 1)

    def _full(shape):
        nd = len(shape)
        return pl.BlockSpec(shape, lambda g, _nd=nd: (0,) * _nd)

    weights = [A, bf,
               Wbn0, bbn0.reshape(1, 64), Wg0, bg0.reshape(1, 32),
               Wbn1, bbn1.reshape(1, 64), Wg1, bg1.reshape(1, 32),
               Wbn2, bbn2.reshape(1, 64), Wg2, bg2.reshape(1, 32),
               Wbn3, bbn3.reshape(1, 64), Wg3, bg3.reshape(1, 32),
               wend_r, bend.reshape(1, 3)]
    in_specs = [pl.BlockSpec((_G, _C, _FEAT), lambda g: (g, 0, 0))]
    in_specs += [_full(w.shape) for w in weights]
    out_specs = [pl.BlockSpec((_G, 1, 3), lambda g: (g, 0, 0))] * 2
    out_shape = [jax.ShapeDtypeStruct((n, 1, 3), jnp.float32)] * 2
    lo, pr = pl.pallas_call(
        _fused_kernel,
        grid=(n // _G,),
        in_specs=in_specs,
        out_specs=out_specs,
        out_shape=out_shape,
    )(xg, *weights)
    return lo.reshape(n, 3), pr.reshape(n, 3)
